# baseline (speedup 1.0000x reference)
import functools

import jax
import jax.numpy as jnp
from jax import lax
from jax.experimental import pallas as pl
from jax.experimental.pallas import tpu as pltpu
from jax.experimental.pallas import tpu_sc as plsc

_L = 128
_NBUF = 4


def _gather_kernel(S, V, D, B0, NC, nb):
    mesh = plsc.VectorSubcoreMesh(core_axis_name="c", subcore_axis_name="s")
    assert S % 2 == 0

    @functools.partial(
        pl.kernel,
        mesh=mesh,
        compiler_params=pltpu.CompilerParams(
            use_tc_tiling_on_sc=True, needs_layout_passes=False
        ),
        out_type=jax.ShapeDtypeStruct((S, D, B0), jnp.float32),
        scratch_types=[
            pltpu.VMEM((S, _L), jnp.int32),
            pltpu.VMEM((S, _L), jnp.int32),
            pltpu.VMEM((_NBUF, _L, 2 * D), jnp.float32),
            pltpu.VMEM((2, D, _L), jnp.float32),
            pltpu.SemaphoreType.DMA,
            pltpu.SemaphoreType.DMA,
        ],
    )
    def k(xT_hbm, tab_hbm, out_hbm, idx_v, pidx_v, rows_v, outb_v, gsem, wsem):
        wid = lax.axis_index("s") * NC + lax.axis_index("c")
        lane = lax.iota(jnp.int32, 16)

        def fire_gather(s, slot):
            for h in range(2):
                pltpu.async_copy(
                    tab_hbm.at[pidx_v.at[s, pl.ds(h * (_L // 2), _L // 2)]],
                    rows_v.at[slot, pl.ds(h * (_L // 2), _L // 2)],
                    gsem,
                )

        def drain_gather(s, slot):
            for h in range(2):
                pltpu.make_async_copy(
                    tab_hbm.at[pidx_v.at[s, pl.ds(h * (_L // 2), _L // 2)]],
                    rows_v.at[slot, pl.ds(h * (_L // 2), _L // 2)],
                    gsem,
                ).wait()

        def shuffle(s, slot, oslot):
            rows = rows_v.at[slot]
            outb = outb_v.at[oslot]

            def lgbody(lg, _):
                l0 = lg * 16
                rvec = lane + l0
                v16 = idx_v[s, pl.ds(l0, 16)]
                h16 = (v16 & 1) * D
                for d in range(D):
                    val = plsc.load_gather(rows, [rvec, h16 + d])
                    outb[d, pl.ds(l0, 16)] = val
                return 0

            lax.fori_loop(0, _L // 16, lgbody, 0)

        def fire_write(s, oslot, bcol):
            pltpu.async_copy(
                outb_v.at[oslot], out_hbm.at[s, :, pl.ds(bcol, _L)], wsem
            )

        def drain_write(s, oslot, bcol):
            pltpu.make_async_copy(
                outb_v.at[oslot], out_hbm.at[s, :, pl.ds(bcol, _L)], wsem
            ).wait()

        def step(s, slot, oslot, bcol):
            @pl.when(s >= 2)
            def _():
                drain_write(s - 2, oslot, bcol)

            @pl.when(s + _NBUF - 1 < S)
            def _():
                fire_gather(s + _NBUF - 1, (slot + _NBUF - 1) % _NBUF)

            drain_gather(s, slot)
            shuffle(s, slot, oslot)
            fire_write(s, oslot, bcol)

        for bi in range(nb):
            bcol = pl.multiple_of((wid * nb + bi) * _L, _L)
            pltpu.sync_copy(xT_hbm.at[:, pl.ds(bcol, _L)], idx_v)

            def pbody(sg, _):
                for lg in range(_L // 16):
                    v16 = idx_v[sg, pl.ds(16 * lg, 16)]
                    pidx_v[sg, pl.ds(16 * lg, 16)] = v16 >> 1
                return 0

            lax.fori_loop(0, S, pbody, 0)

            for s in range(_NBUF - 1):
                fire_gather(s, s % _NBUF)

            def body(h, _):
                s0 = h * _NBUF
                for j in range(_NBUF):
                    step(s0 + j, j, j % 2, bcol)
                return 0

            n_full = (S - 2) // _NBUF
            lax.fori_loop(0, n_full, body, 0)
            step(S - 2, (S - 2) % _NBUF, 0, bcol)
            step(S - 1, (S - 1) % _NBUF, 1, bcol)
            drain_write(S - 2, 0, bcol)
            drain_write(S - 1, 1, bcol)

    return k


def kernel(x, table):
    B0, S = x.shape
    V, D = table.shape

    info = plsc.get_sparse_core_info()
    NC, NS = info.num_cores, info.num_subcores
    NW = NC * NS
    assert B0 % (NW * _L) == 0
    nb = B0 // (NW * _L)
    assert (S - 2) % _NBUF == 0

    xT = x.T.astype(jnp.int32)
    tab2 = table.reshape(V // 2, 2 * D)
    k = _gather_kernel(S, V, D, B0, NC, nb)
    out3 = k(xT, tab2)
    return out3.transpose(2, 0, 1)


# speedup vs baseline: 1.1514x; 1.1514x over previous
import functools

import jax
import jax.numpy as jnp
from jax import lax
from jax.experimental import pallas as pl
from jax.experimental.pallas import tpu as pltpu
from jax.experimental.pallas import tpu_sc as plsc

_L = 128
_NBUF = 4


def _gather_kernel(S, V, D, B0, NC, nb):
    mesh = plsc.VectorSubcoreMesh(core_axis_name="c", subcore_axis_name="s")
    TA, TR = D // 8, 8

    @functools.partial(
        pl.kernel,
        mesh=mesh,
        compiler_params=pltpu.CompilerParams(
            use_tc_tiling_on_sc=False, needs_layout_passes=False
        ),
        out_type=jax.ShapeDtypeStruct((S, TA, B0 // _L, TR, _L), jnp.float32),
        scratch_types=[
            pltpu.VMEM((S, _L), jnp.int32),
            pltpu.VMEM((_NBUF, _L, D), jnp.float32),
            pltpu.VMEM((2, TA, TR, _L), jnp.float32),
            pltpu.SemaphoreType.DMA,
            pltpu.SemaphoreType.DMA,
        ],
    )
    def k(xT_hbm, tab_hbm, out_hbm, idx_v, rows_v, outb_v, gsem, wsem):
        wid = lax.axis_index("s") * NC + lax.axis_index("c")
        lane = lax.iota(jnp.int32, 16)
        zero16 = lane - lane
        avecs = [lane // 8 + (2 * j) for j in range(D // 16)]
        rvecs = [(lane % 8) for _ in range(D // 16)]

        def fire_gather(s, slot):
            pltpu.async_copy(tab_hbm.at[idx_v.at[s]], rows_v.at[slot], gsem)

        def drain_gather(s, slot):
            pltpu.make_async_copy(
                tab_hbm.at[idx_v.at[s]], rows_v.at[slot], gsem
            ).wait()

        def shuffle(s, slot, oslot):
            rows = rows_v.at[slot]
            outb = outb_v.at[oslot]

            def lgbody(lg, _):
                l0 = lg * 16
                for li in range(16):
                    l = l0 + li
                    lvec = zero16 + l
                    for j in range(D // 16):
                        val = rows[l, pl.ds(16 * j, 16)]
                        plsc.store_scatter(outb, [avecs[j], rvecs[j], lvec], val)
                return 0

            lax.fori_loop(0, _L // 16, lgbody, 0)

        def fire_write(s, oslot, blk):
            pltpu.async_copy(
                outb_v.at[oslot], out_hbm.at[s, :, blk, :, :], wsem
            )

        def drain_write(s, oslot, blk):
            pltpu.make_async_copy(
                outb_v.at[oslot], out_hbm.at[s, :, blk, :, :], wsem
            ).wait()

        def step(s, slot, oslot, blk):
            @pl.when(s >= 2)
            def _():
                drain_write(s - 2, oslot, blk)

            @pl.when(s + _NBUF - 1 < S)
            def _():
                fire_gather(s + _NBUF - 1, (slot + _NBUF - 1) % _NBUF)

            drain_gather(s, slot)
            shuffle(s, slot, oslot)
            fire_write(s, oslot, blk)

        def bibody(bi, _):
            blk = wid * nb + bi
            bcol = pl.multiple_of(blk * _L, _L)
            pltpu.sync_copy(xT_hbm.at[:, pl.ds(bcol, _L)], idx_v)

            for s in range(_NBUF - 1):
                fire_gather(s, s % _NBUF)

            def body(h, _):
                s0 = h * _NBUF
                for j in range(_NBUF):
                    step(s0 + j, j, j % 2, blk)
                return 0

            n_full = (S - 2) // _NBUF
            lax.fori_loop(0, n_full, body, 0)
            step(S - 2, (S - 2) % _NBUF, 0, blk)
            step(S - 1, (S - 1) % _NBUF, 1, blk)
            drain_write(S - 2, 0, blk)
            drain_write(S - 1, 1, blk)
            return 0

        lax.fori_loop(0, nb, bibody, 0)

    return k


def kernel(x, table):
    B0, S = x.shape
    V, D = table.shape

    info = plsc.get_sparse_core_info()
    NC, NS = info.num_cores, info.num_subcores
    NW = NC * NS
    assert B0 % (NW * _L) == 0
    nb = B0 // (NW * _L)
    assert (S - 2) % _NBUF == 0

    xT = x.T.astype(jnp.int32)
    k = _gather_kernel(S, V, D, B0, NC, nb)
    out5 = k(xT, table)
    return out5.transpose(2, 4, 0, 1, 3).reshape(B0, S, D)
